# baseline (speedup 1.0000x reference)
import functools

import jax
import jax.numpy as jnp
from jax import lax
from jax.experimental import pallas as pl
from jax.experimental.pallas import tpu as pltpu
from jax.experimental.pallas import tpu_sc as plsc

N = 10000
E = 320000
HID = 128
H2 = HID // 2
NC = 2
NS = 16
CHUNK = 128
NCHUNKS = E // CHUNK
SLAB = 632
NPAD = SLAB * NS
RB = 2560


def _embed_h_body(h_ref, p_ref, wh_ref, bh_ref, wp_ref, bp_ref, o_ref):
    o_ref[...] = (
        jnp.dot(h_ref[...], wh_ref[...], preferred_element_type=jnp.float32)
        + jnp.dot(p_ref[...], wp_ref[...], preferred_element_type=jnp.float32)
        + bh_ref[...] + bp_ref[...]
    )


def _embed_e_body(e_ref, we_ref, be_ref, o_ref):
    o_ref[...] = (
        jnp.dot(e_ref[...], we_ref[...], preferred_element_type=jnp.float32)
        + be_ref[...]
    )


def _nodemm_body(h_ref, w_ref, b_ref, ah_ref, eb_ref, d_ref):
    r = jnp.dot(h_ref[...], w_ref[...], preferred_element_type=jnp.float32)
    r = r + b_ref[...]
    ah_ref[...] = r[:, 0:128]
    eb_ref[0] = jnp.concatenate([r[:, 384:448], r[:, 128:192]], axis=1)
    eb_ref[1] = jnp.concatenate([r[:, 448:512], r[:, 192:256]], axis=1)
    d_ref[0] = r[:, 256:320]
    d_ref[1] = r[:, 320:384]


def _ce0_body(e_ref, w_ref, b_ref, ce_ref):
    r = jnp.dot(e_ref[...], w_ref[...], preferred_element_type=jnp.float32)
    r = r + b_ref[...]
    ce_ref[0] = r[:, :H2]
    ce_ref[1] = r[:, H2:]


def _ce_upd_body(write_e, e_ref, en_ref, aff_ref, w_ref, b_ref, *outs):
    if write_e:
        eo_ref, ce_ref = outs
    else:
        (ce_ref,) = outs
    en = jnp.concatenate([en_ref[0], en_ref[1]], axis=1)
    e = e_ref[...] + jnp.maximum(en * aff_ref[0:1, :] + aff_ref[1:2, :], 0.0)
    if write_e:
        eo_ref[...] = e
    r = jnp.dot(e, w_ref[...], preferred_element_type=jnp.float32) + b_ref[...]
    ce_ref[0] = r[:, :H2]
    ce_ref[1] = r[:, H2:]


def _nodeupd_body(last, *refs):
    if last:
        ah_ref, num_ref, den_ref, h_ref, g_ref, b_ref, ho_ref = refs
    else:
        (ah_ref, num_ref, den_ref, h_ref, g_ref, b_ref, bn_ref, ge_ref,
         be_ref, ho_ref, aff_ref) = refs
    num = jnp.concatenate([num_ref[0][:N], num_ref[1][:N]], axis=1)
    den = jnp.concatenate([den_ref[0][:N], den_ref[1][:N]], axis=1)
    hn = ah_ref[...] + num / (den + 1e-6)
    mu = jnp.mean(hn, axis=0, keepdims=True)
    var = jnp.mean((hn - mu) ** 2, axis=0, keepdims=True)
    hnn = (hn - mu) / jnp.sqrt(var + 1e-5) * g_ref[...] + b_ref[...]
    ho_ref[...] = h_ref[...] + jnp.maximum(hnn, 0.0)
    if not last:
        s = jnp.concatenate(
            [jnp.sum(bn_ref[0:NS, 0:H2], axis=0, keepdims=True),
             jnp.sum(bn_ref[NS:2 * NS, 0:H2], axis=0, keepdims=True)], axis=1)
        q = jnp.concatenate(
            [jnp.sum(bn_ref[0:NS, H2:HID], axis=0, keepdims=True),
             jnp.sum(bn_ref[NS:2 * NS, H2:HID], axis=0, keepdims=True)], axis=1)
        mu_e = s / E
        var_e = q / E - mu_e * mu_e
        sc = ge_ref[...] / jnp.sqrt(var_e + 1e-5)
        sh = be_ref[...] - mu_e * sc
        aff_ref[...] = jnp.concatenate([sc, sh], axis=0)


def _readout_body(h_ref, w1_ref, b1_ref, w2_ref, b2_ref, w3_ref, b3_ref, o_ref):
    hg = jnp.mean(h_ref[...], axis=0, keepdims=True)
    x = jnp.maximum(
        jnp.dot(hg, w1_ref[...], preferred_element_type=jnp.float32)
        + b1_ref[...], 0.0)
    x = jnp.maximum(
        jnp.dot(x, w2_ref[...], preferred_element_type=jnp.float32)
        + b2_ref[...], 0.0)
    o_ref[...] = (jnp.dot(x, w3_ref[...], preferred_element_type=jnp.float32)
                  + b3_ref[...])


_embed_h = pl.pallas_call(
    _embed_h_body,
    out_shape=jax.ShapeDtypeStruct((N, HID), jnp.float32),
)

_embed_e = pl.pallas_call(
    _embed_e_body,
    grid=(E // RB,),
    in_specs=[
        pl.BlockSpec((RB, 16), lambda i: (i, 0)),
        pl.BlockSpec((16, HID), lambda i: (0, 0)),
        pl.BlockSpec((1, HID), lambda i: (0, 0)),
    ],
    out_specs=pl.BlockSpec((RB, HID), lambda i: (i, 0)),
    out_shape=jax.ShapeDtypeStruct((E, HID), jnp.float32),
)

_nodemm = pl.pallas_call(
    _nodemm_body,
    out_shape=[
        jax.ShapeDtypeStruct((N, HID), jnp.float32),
        jax.ShapeDtypeStruct((2, N, HID), jnp.float32),
        jax.ShapeDtypeStruct((2, N, H2), jnp.float32),
    ],
)

_ce0 = pl.pallas_call(
    _ce0_body,
    grid=(E // RB,),
    in_specs=[
        pl.BlockSpec((RB, HID), lambda i: (i, 0)),
        pl.BlockSpec((HID, HID), lambda i: (0, 0)),
        pl.BlockSpec((1, HID), lambda i: (0, 0)),
    ],
    out_specs=pl.BlockSpec((2, RB, H2), lambda i: (0, i, 0)),
    out_shape=jax.ShapeDtypeStruct((2, E, H2), jnp.float32),
)


def _make_ce_upd(write_e):
    out_specs = [pl.BlockSpec((2, RB, H2), lambda i: (0, i, 0))]
    out_shape = [jax.ShapeDtypeStruct((2, E, H2), jnp.float32)]
    if write_e:
        out_specs = [pl.BlockSpec((RB, HID), lambda i: (i, 0))] + out_specs
        out_shape = [jax.ShapeDtypeStruct((E, HID), jnp.float32)] + out_shape
    return pl.pallas_call(
        functools.partial(_ce_upd_body, write_e),
        grid=(E // RB,),
        in_specs=[
            pl.BlockSpec((RB, HID), lambda i: (i, 0)),
            pl.BlockSpec((2, RB, H2), lambda i: (0, i, 0)),
            pl.BlockSpec((2, HID), lambda i: (0, 0)),
            pl.BlockSpec((HID, HID), lambda i: (0, 0)),
            pl.BlockSpec((1, HID), lambda i: (0, 0)),
        ],
        out_specs=out_specs,
        out_shape=out_shape,
    )


_ce_upd = _make_ce_upd(True)
_ce_last = _make_ce_upd(False)

_nodeupd = pl.pallas_call(
    functools.partial(_nodeupd_body, False),
    out_shape=[
        jax.ShapeDtypeStruct((N, HID), jnp.float32),
        jax.ShapeDtypeStruct((2, HID), jnp.float32),
    ],
)

_nodeupd_last = pl.pallas_call(
    functools.partial(_nodeupd_body, True),
    out_shape=jax.ShapeDtypeStruct((N, HID), jnp.float32),
)

_readout = pl.pallas_call(
    _readout_body,
    out_shape=jax.ShapeDtypeStruct((1, 1), jnp.float32),
)


def _sc_edge_body(*refs):
    (eb_hbm, d_hbm, ce_hbm, srcx_hbm, dstx_hbm, dst_hbm, zeros_hbm,
     enew_hbm, num_hbm, den_hbm, bn_hbm,
     idx_s, idx_d2, idx_d, eb_v, d_v, ce_v, en_v, sc_v, bnacc,
     sem1, sem2, tbl) = refs

    c = lax.axis_index("c")
    s = lax.axis_index("s")
    row0 = s * SLAB

    def zero_tbl():
        pltpu.sync_copy(zeros_hbm.at[pl.ds(row0, SLAB)],
                        tbl.at[pl.ds(row0, SLAB)])

    zero_tbl()
    for k in range(HID // 16):
        bnacc[pl.ds(k * 16, 16)] = jnp.zeros((16,), jnp.float32)
    plsc.subcore_barrier()

    ntrips = (NCHUNKS - 1 - s) // NS + 1

    def trip_a(j, carry):
        chunk = s + j * NS
        base = chunk * CHUNK
        pltpu.sync_copy(srcx_hbm.at[pl.ds(c * E + base, CHUNK)], idx_s)
        pltpu.sync_copy(dstx_hbm.at[pl.ds(c * E + base, CHUNK)], idx_d2)
        pltpu.sync_copy(dst_hbm.at[pl.ds(base, CHUNK)], idx_d)
        g1 = pltpu.async_copy(eb_hbm.at[idx_s], eb_v, sem1)
        g2 = pltpu.async_copy(d_hbm.at[idx_d2], d_v, sem2)
        pltpu.sync_copy(ce_hbm.at[pl.ds(c * E + base, CHUNK)], ce_v)
        g1.wait()
        g2.wait()

        def row(r, cr):
            for k in range(H2 // 16):
                sl = pl.ds(k * 16, 16)
                sl2 = pl.ds(H2 + k * 16, 16)
                en = d_v[r, sl] + eb_v[r, sl] + ce_v[r, sl]
                sg = 1.0 / (1.0 + jnp.exp(-en))
                sc_v[r, sl] = sg * eb_v[r, sl2]
                en_v[r, sl] = en
                bnacc[sl] = bnacc[sl] + en
                bnacc[sl2] = bnacc[sl2] + en * en
            return cr

        lax.fori_loop(0, CHUNK, row, 0, unroll=False)
        pltpu.sync_copy(en_v, enew_hbm.at[pl.ds(c * E + base, CHUNK)])
        pltpu.sync_copy(sc_v, tbl.at[idx_d], add=True)
        return carry

    lax.fori_loop(0, ntrips, trip_a, 0, unroll=False)

    plsc.subcore_barrier()
    pltpu.sync_copy(tbl.at[pl.ds(row0, SLAB)],
                    num_hbm.at[pl.ds(c * NPAD + row0, SLAB)])
    w = c * NS + s
    pltpu.sync_copy(bnacc, bn_hbm.at[pl.ds(w * HID, HID)])
    zero_tbl()
    plsc.subcore_barrier()

    def trip_b(j, carry):
        chunk = s + j * NS
        base = chunk * CHUNK
        pltpu.sync_copy(dst_hbm.at[pl.ds(base, CHUNK)], idx_d)
        pltpu.sync_copy(enew_hbm.at[pl.ds(c * E + base, CHUNK)], ce_v)

        def row(r, cr):
            for k in range(H2 // 16):
                sl = pl.ds(k * 16, 16)
                sc_v[r, sl] = 1.0 / (1.0 + jnp.exp(-ce_v[r, sl]))
            return cr

        lax.fori_loop(0, CHUNK, row, 0, unroll=False)
        pltpu.sync_copy(sc_v, tbl.at[idx_d], add=True)
        return carry

    lax.fori_loop(0, ntrips, trip_b, 0, unroll=False)

    plsc.subcore_barrier()
    pltpu.sync_copy(tbl.at[pl.ds(row0, SLAB)],
                    den_hbm.at[pl.ds(c * NPAD + row0, SLAB)])


@functools.lru_cache(maxsize=None)
def _make_sc_edge():
    mesh = plsc.VectorSubcoreMesh(core_axis_name="c", subcore_axis_name="s",
                                  num_cores=NC, num_subcores=NS)
    out_type = [
        jax.ShapeDtypeStruct((2 * E, H2), jnp.float32),
        jax.ShapeDtypeStruct((2 * NPAD, H2), jnp.float32),
        jax.ShapeDtypeStruct((2 * NPAD, H2), jnp.float32),
        jax.ShapeDtypeStruct((2 * NS * HID,), jnp.float32),
    ]
    scratch = [
        pltpu.VMEM((CHUNK,), jnp.int32),
        pltpu.VMEM((CHUNK,), jnp.int32),
        pltpu.VMEM((CHUNK,), jnp.int32),
        pltpu.VMEM((CHUNK, HID), jnp.float32),
        pltpu.VMEM((CHUNK, H2), jnp.float32),
        pltpu.VMEM((CHUNK, H2), jnp.float32),
        pltpu.VMEM((CHUNK, H2), jnp.float32),
        pltpu.VMEM((CHUNK, H2), jnp.float32),
        pltpu.VMEM((HID,), jnp.float32),
        pltpu.SemaphoreType.DMA,
        pltpu.SemaphoreType.DMA,
        pltpu.VMEM_SHARED((NPAD, H2), jnp.float32),
    ]
    return pl.kernel(
        _sc_edge_body,
        out_type,
        mesh=mesh,
        scratch_types=scratch,
        compiler_params=pltpu.CompilerParams(use_tc_tiling_on_sc=False),
    )


def _sc_edge(*args):
    return _make_sc_edge()(*args)


def kernel(h, e, p, edge_index, W_h, b_h, W_p, b_p, W_e, b_e,
           lA, lbA, lB, lbB, lC, lbC, lD, lbD, lE, lbE,
           g_h, be_h, g_e, be_e, W1, b1, W2, b2, W3, b3):
    src = edge_index[0]
    dst = edge_index[1]
    srcx = jnp.concatenate([src, src + N])
    dstx = jnp.concatenate([dst, dst + N])
    zeros_nd = jnp.zeros((NPAD, H2), jnp.float32)

    hcur = _embed_h(h, p, W_h, b_h.reshape(1, -1), W_p, b_p.reshape(1, -1))
    e0 = _embed_e(e, W_e, b_e.reshape(1, -1))

    e_base = e0
    enew_prev = None
    aff = None
    NL = lA.shape[0]
    for i in range(NL):
        wcat = jnp.concatenate([lA[i], lB[i], lD[i], lE[i]], axis=1)
        bcat = jnp.concatenate([lbA[i], lbB[i], lbD[i], lbE[i]]).reshape(1, -1)
        ah, ebt, dt = _nodemm(hcur, wcat, bcat)
        if i == 0:
            ce = _ce0(e_base, lC[i], lbC[i].reshape(1, -1))
        elif i < NL - 1:
            e_base, ce = _ce_upd(e_base, enew_prev, aff, lC[i],
                                 lbC[i].reshape(1, -1))
        else:
            (ce,) = _ce_last(e_base, enew_prev, aff, lC[i],
                             lbC[i].reshape(1, -1))
        ebf = ebt.reshape(2 * N, HID)
        df = dt.reshape(2 * N, H2)
        cef = ce.reshape(2 * E, H2)
        enew, num, den, bnp = _sc_edge(ebf, df, cef, srcx, dstx, dst,
                                       zeros_nd)
        num2 = num.reshape(2, NPAD, H2)
        den2 = den.reshape(2, NPAD, H2)
        if i < NL - 1:
            hcur, aff = _nodeupd(ah, num2, den2, hcur,
                                 g_h[i].reshape(1, -1), be_h[i].reshape(1, -1),
                                 bnp.reshape(2 * NS, HID), g_e[i].reshape(1, -1),
                                 be_e[i].reshape(1, -1))
            enew_prev = enew.reshape(2, E, H2)
        else:
            hcur = _nodeupd_last(ah, num2, den2, hcur,
                                 g_h[i].reshape(1, -1),
                                 be_h[i].reshape(1, -1))

    return _readout(hcur, W1, b1.reshape(1, -1), W2, b2.reshape(1, -1),
                    W3, b3.reshape(1, -1))


# speedup vs baseline: 1.2142x; 1.2142x over previous
import functools

import jax
import jax.numpy as jnp
from jax import lax
from jax.experimental import pallas as pl
from jax.experimental.pallas import tpu as pltpu
from jax.experimental.pallas import tpu_sc as plsc

N = 10000
E = 320000
HID = 128
H2 = HID // 2
NC = 2
NS = 16
CHUNK = 128
NCHUNKS = E // CHUNK
SLAB = 632
NPAD = SLAB * NS
RB = 2560


def _embed_h_body(h_ref, p_ref, wh_ref, bh_ref, wp_ref, bp_ref, o_ref):
    o_ref[...] = (
        jnp.dot(h_ref[...], wh_ref[...], preferred_element_type=jnp.float32)
        + jnp.dot(p_ref[...], wp_ref[...], preferred_element_type=jnp.float32)
        + bh_ref[...] + bp_ref[...]
    )


def _embed_e_body(e_ref, we_ref, be_ref, o_ref):
    o_ref[...] = (
        jnp.dot(e_ref[...], we_ref[...], preferred_element_type=jnp.float32)
        + be_ref[...]
    )


def _nodemm_body(h_ref, w_ref, b_ref, ah_ref, eb_ref, d_ref):
    r = jnp.dot(h_ref[...], w_ref[...], preferred_element_type=jnp.float32)
    r = r + b_ref[...]
    ah_ref[...] = r[:, 0:128]
    eb_ref[0] = jnp.concatenate([r[:, 384:448], r[:, 128:192]], axis=1)
    eb_ref[1] = jnp.concatenate([r[:, 448:512], r[:, 192:256]], axis=1)
    d_ref[0] = r[:, 256:320]
    d_ref[1] = r[:, 320:384]


def _ce0_body(e_ref, w_ref, b_ref, ce_ref):
    r = jnp.dot(e_ref[...], w_ref[...], preferred_element_type=jnp.float32)
    r = r + b_ref[...]
    ce_ref[0] = r[:, :H2]
    ce_ref[1] = r[:, H2:]


def _ce_upd_body(write_e, e_ref, en_ref, aff_ref, w_ref, b_ref, *outs):
    if write_e:
        eo_ref, ce_ref = outs
    else:
        (ce_ref,) = outs
    en = jnp.concatenate([en_ref[0], en_ref[1]], axis=1)
    e = e_ref[...] + jnp.maximum(en * aff_ref[0:1, :] + aff_ref[1:2, :], 0.0)
    if write_e:
        eo_ref[...] = e
    r = jnp.dot(e, w_ref[...], preferred_element_type=jnp.float32) + b_ref[...]
    ce_ref[0] = r[:, :H2]
    ce_ref[1] = r[:, H2:]


def _nodeupd_body(last, *refs):
    if last:
        ah_ref, num_ref, den_ref, h_ref, g_ref, b_ref, ho_ref = refs
    else:
        (ah_ref, num_ref, den_ref, h_ref, g_ref, b_ref, bn_ref, ge_ref,
         be_ref, ho_ref, aff_ref) = refs
    num = jnp.concatenate([num_ref[0][:N], num_ref[1][:N]], axis=1)
    den = jnp.concatenate([den_ref[0][:N], den_ref[1][:N]], axis=1)
    hn = ah_ref[...] + num / (den + 1e-6)
    mu = jnp.mean(hn, axis=0, keepdims=True)
    var = jnp.mean((hn - mu) ** 2, axis=0, keepdims=True)
    hnn = (hn - mu) / jnp.sqrt(var + 1e-5) * g_ref[...] + b_ref[...]
    ho_ref[...] = h_ref[...] + jnp.maximum(hnn, 0.0)
    if not last:
        s = jnp.concatenate(
            [jnp.sum(bn_ref[0:NS, 0:H2], axis=0, keepdims=True),
             jnp.sum(bn_ref[NS:2 * NS, 0:H2], axis=0, keepdims=True)], axis=1)
        q = jnp.concatenate(
            [jnp.sum(bn_ref[0:NS, H2:HID], axis=0, keepdims=True),
             jnp.sum(bn_ref[NS:2 * NS, H2:HID], axis=0, keepdims=True)], axis=1)
        mu_e = s / E
        var_e = q / E - mu_e * mu_e
        sc = ge_ref[...] / jnp.sqrt(var_e + 1e-5)
        sh = be_ref[...] - mu_e * sc
        aff_ref[...] = jnp.concatenate([sc, sh], axis=0)


def _readout_body(h_ref, w1_ref, b1_ref, w2_ref, b2_ref, w3_ref, b3_ref, o_ref):
    hg = jnp.mean(h_ref[...], axis=0, keepdims=True)
    x = jnp.maximum(
        jnp.dot(hg, w1_ref[...], preferred_element_type=jnp.float32)
        + b1_ref[...], 0.0)
    x = jnp.maximum(
        jnp.dot(x, w2_ref[...], preferred_element_type=jnp.float32)
        + b2_ref[...], 0.0)
    o_ref[...] = (jnp.dot(x, w3_ref[...], preferred_element_type=jnp.float32)
                  + b3_ref[...])


_embed_h = pl.pallas_call(
    _embed_h_body,
    out_shape=jax.ShapeDtypeStruct((N, HID), jnp.float32),
)

_embed_e = pl.pallas_call(
    _embed_e_body,
    grid=(E // RB,),
    in_specs=[
        pl.BlockSpec((RB, 16), lambda i: (i, 0)),
        pl.BlockSpec((16, HID), lambda i: (0, 0)),
        pl.BlockSpec((1, HID), lambda i: (0, 0)),
    ],
    out_specs=pl.BlockSpec((RB, HID), lambda i: (i, 0)),
    out_shape=jax.ShapeDtypeStruct((E, HID), jnp.float32),
)

_nodemm = pl.pallas_call(
    _nodemm_body,
    out_shape=[
        jax.ShapeDtypeStruct((N, HID), jnp.float32),
        jax.ShapeDtypeStruct((2, N, HID), jnp.float32),
        jax.ShapeDtypeStruct((2, N, H2), jnp.float32),
    ],
)

_ce0 = pl.pallas_call(
    _ce0_body,
    grid=(E // RB,),
    in_specs=[
        pl.BlockSpec((RB, HID), lambda i: (i, 0)),
        pl.BlockSpec((HID, HID), lambda i: (0, 0)),
        pl.BlockSpec((1, HID), lambda i: (0, 0)),
    ],
    out_specs=pl.BlockSpec((2, RB, H2), lambda i: (0, i, 0)),
    out_shape=jax.ShapeDtypeStruct((2, E, H2), jnp.float32),
)


def _make_ce_upd(write_e):
    out_specs = [pl.BlockSpec((2, RB, H2), lambda i: (0, i, 0))]
    out_shape = [jax.ShapeDtypeStruct((2, E, H2), jnp.float32)]
    if write_e:
        out_specs = [pl.BlockSpec((RB, HID), lambda i: (i, 0))] + out_specs
        out_shape = [jax.ShapeDtypeStruct((E, HID), jnp.float32)] + out_shape
    return pl.pallas_call(
        functools.partial(_ce_upd_body, write_e),
        grid=(E // RB,),
        in_specs=[
            pl.BlockSpec((RB, HID), lambda i: (i, 0)),
            pl.BlockSpec((2, RB, H2), lambda i: (0, i, 0)),
            pl.BlockSpec((2, HID), lambda i: (0, 0)),
            pl.BlockSpec((HID, HID), lambda i: (0, 0)),
            pl.BlockSpec((1, HID), lambda i: (0, 0)),
        ],
        out_specs=out_specs,
        out_shape=out_shape,
    )


_ce_upd = _make_ce_upd(True)
_ce_last = _make_ce_upd(False)

_nodeupd = pl.pallas_call(
    functools.partial(_nodeupd_body, False),
    out_shape=[
        jax.ShapeDtypeStruct((N, HID), jnp.float32),
        jax.ShapeDtypeStruct((2, HID), jnp.float32),
    ],
)

_nodeupd_last = pl.pallas_call(
    functools.partial(_nodeupd_body, True),
    out_shape=jax.ShapeDtypeStruct((N, HID), jnp.float32),
)

_readout = pl.pallas_call(
    _readout_body,
    out_shape=jax.ShapeDtypeStruct((1, 1), jnp.float32),
)


def _sc_edge_body(*refs):
    (eb_hbm, d_hbm, ce_hbm, srcx_hbm, dstx_hbm, dst_hbm, zeros_hbm,
     enew_hbm, sg_hbm, num_hbm, den_hbm, bn_hbm,
     idx_s, idx_d2, idx_d, eb_v, d_v, ce_v, en_v, sc_v, bnacc,
     sem1, sem2, sem3, sem4, sem5, tbl) = refs

    c = lax.axis_index("c")
    s = lax.axis_index("s")
    row0 = s * SLAB

    def zero_tbl():
        pltpu.sync_copy(zeros_hbm.at[pl.ds(row0, SLAB)],
                        tbl.at[pl.ds(row0, SLAB)])

    zero_tbl()
    plsc.subcore_barrier()

    ntrips = (NCHUNKS - 1 - s) // NS + 1

    def trip_a(j, bn):
        chunk = s + j * NS
        base = chunk * CHUNK
        cp_s = pltpu.async_copy(srcx_hbm.at[pl.ds(c * E + base, CHUNK)],
                                idx_s, sem1)
        cp_x = pltpu.async_copy(dstx_hbm.at[pl.ds(c * E + base, CHUNK)],
                                idx_d2, sem2)
        cp_d = pltpu.async_copy(dst_hbm.at[pl.ds(base, CHUNK)], idx_d, sem3)
        cp_c = pltpu.async_copy(ce_hbm.at[pl.ds(c * E + base, CHUNK)],
                                ce_v, sem4)
        cp_s.wait()
        g1 = pltpu.async_copy(eb_hbm.at[idx_s], eb_v, sem1)
        cp_x.wait()
        g2 = pltpu.async_copy(d_hbm.at[idx_d2], d_v, sem2)
        cp_c.wait()
        g1.wait()
        g2.wait()

        def row(r, bn2):
            acc = list(bn2)
            for k in range(H2 // 16):
                sl = pl.ds(k * 16, 16)
                sl2 = pl.ds(H2 + k * 16, 16)
                en = d_v[r, sl] + eb_v[r, sl] + ce_v[r, sl]
                sg = 1.0 / (1.0 + jnp.exp(-en))
                sc_v[r, sl] = sg * eb_v[r, sl2]
                en_v[r, sl] = en
                ce_v[r, sl] = sg
                acc[k] = acc[k] + en
                acc[4 + k] = acc[4 + k] + en * en
            return tuple(acc)

        bn = lax.fori_loop(0, CHUNK, row, bn, unroll=4)
        w_en = pltpu.async_copy(en_v, enew_hbm.at[pl.ds(c * E + base, CHUNK)],
                                sem4)
        w_sg = pltpu.async_copy(ce_v, sg_hbm.at[pl.ds(c * E + base, CHUNK)],
                                sem5)
        cp_d.wait()
        pltpu.sync_copy(sc_v, tbl.at[idx_d], add=True)
        w_en.wait()
        w_sg.wait()
        return bn

    bn0 = tuple(jnp.zeros((16,), jnp.float32) for _ in range(8))
    bn = lax.fori_loop(0, ntrips, trip_a, bn0, unroll=False)
    for k in range(8):
        bnacc[pl.ds(k * 16, 16)] = bn[k]

    plsc.subcore_barrier()
    pltpu.sync_copy(tbl.at[pl.ds(row0, SLAB)],
                    num_hbm.at[pl.ds(c * NPAD + row0, SLAB)])
    w = c * NS + s
    pltpu.sync_copy(bnacc, bn_hbm.at[pl.ds(w * HID, HID)])
    zero_tbl()
    plsc.subcore_barrier()

    def trip_b(j, carry):
        chunk = s + j * NS
        base = chunk * CHUNK
        cp_d = pltpu.async_copy(dst_hbm.at[pl.ds(base, CHUNK)], idx_d, sem3)
        pltpu.async_copy(sg_hbm.at[pl.ds(c * E + base, CHUNK)], sc_v,
                         sem4).wait()
        cp_d.wait()
        pltpu.sync_copy(sc_v, tbl.at[idx_d], add=True)
        return carry

    lax.fori_loop(0, ntrips, trip_b, 0, unroll=False)

    plsc.subcore_barrier()
    pltpu.sync_copy(tbl.at[pl.ds(row0, SLAB)],
                    den_hbm.at[pl.ds(c * NPAD + row0, SLAB)])


@functools.lru_cache(maxsize=None)
def _make_sc_edge():
    mesh = plsc.VectorSubcoreMesh(core_axis_name="c", subcore_axis_name="s",
                                  num_cores=NC, num_subcores=NS)
    out_type = [
        jax.ShapeDtypeStruct((2 * E, H2), jnp.float32),
        jax.ShapeDtypeStruct((2 * E, H2), jnp.float32),
        jax.ShapeDtypeStruct((2 * NPAD, H2), jnp.float32),
        jax.ShapeDtypeStruct((2 * NPAD, H2), jnp.float32),
        jax.ShapeDtypeStruct((2 * NS * HID,), jnp.float32),
    ]
    scratch = [
        pltpu.VMEM((CHUNK,), jnp.int32),
        pltpu.VMEM((CHUNK,), jnp.int32),
        pltpu.VMEM((CHUNK,), jnp.int32),
        pltpu.VMEM((CHUNK, HID), jnp.float32),
        pltpu.VMEM((CHUNK, H2), jnp.float32),
        pltpu.VMEM((CHUNK, H2), jnp.float32),
        pltpu.VMEM((CHUNK, H2), jnp.float32),
        pltpu.VMEM((CHUNK, H2), jnp.float32),
        pltpu.VMEM((HID,), jnp.float32),
        pltpu.SemaphoreType.DMA,
        pltpu.SemaphoreType.DMA,
        pltpu.SemaphoreType.DMA,
        pltpu.SemaphoreType.DMA,
        pltpu.SemaphoreType.DMA,
        pltpu.VMEM_SHARED((NPAD, H2), jnp.float32),
    ]
    return pl.kernel(
        _sc_edge_body,
        out_type,
        mesh=mesh,
        scratch_types=scratch,
        compiler_params=pltpu.CompilerParams(use_tc_tiling_on_sc=False),
    )


def _sc_edge(*args):
    return _make_sc_edge()(*args)


def kernel(h, e, p, edge_index, W_h, b_h, W_p, b_p, W_e, b_e,
           lA, lbA, lB, lbB, lC, lbC, lD, lbD, lE, lbE,
           g_h, be_h, g_e, be_e, W1, b1, W2, b2, W3, b3):
    src = edge_index[0]
    dst = edge_index[1]
    srcx = jnp.concatenate([src, src + N])
    dstx = jnp.concatenate([dst, dst + N])
    zeros_nd = jnp.zeros((NPAD, H2), jnp.float32)

    hcur = _embed_h(h, p, W_h, b_h.reshape(1, -1), W_p, b_p.reshape(1, -1))
    e0 = _embed_e(e, W_e, b_e.reshape(1, -1))

    e_base = e0
    enew_prev = None
    aff = None
    NL = lA.shape[0]
    for i in range(NL):
        wcat = jnp.concatenate([lA[i], lB[i], lD[i], lE[i]], axis=1)
        bcat = jnp.concatenate([lbA[i], lbB[i], lbD[i], lbE[i]]).reshape(1, -1)
        ah, ebt, dt = _nodemm(hcur, wcat, bcat)
        if i == 0:
            ce = _ce0(e_base, lC[i], lbC[i].reshape(1, -1))
        elif i < NL - 1:
            e_base, ce = _ce_upd(e_base, enew_prev, aff, lC[i],
                                 lbC[i].reshape(1, -1))
        else:
            (ce,) = _ce_last(e_base, enew_prev, aff, lC[i],
                             lbC[i].reshape(1, -1))
        ebf = ebt.reshape(2 * N, HID)
        df = dt.reshape(2 * N, H2)
        cef = ce.reshape(2 * E, H2)
        enew, _sg, num, den, bnp = _sc_edge(ebf, df, cef, srcx, dstx, dst,
                                            zeros_nd)
        num2 = num.reshape(2, NPAD, H2)
        den2 = den.reshape(2, NPAD, H2)
        if i < NL - 1:
            hcur, aff = _nodeupd(ah, num2, den2, hcur,
                                 g_h[i].reshape(1, -1), be_h[i].reshape(1, -1),
                                 bnp.reshape(2 * NS, HID), g_e[i].reshape(1, -1),
                                 be_e[i].reshape(1, -1))
            enew_prev = enew.reshape(2, E, H2)
        else:
            hcur = _nodeupd_last(ah, num2, den2, hcur,
                                 g_h[i].reshape(1, -1),
                                 be_h[i].reshape(1, -1))

    return _readout(hcur, W1, b1.reshape(1, -1), W2, b2.reshape(1, -1),
                    W3, b3.reshape(1, -1))
